# baseline (speedup 1.0000x reference)
import functools

import jax
import jax.numpy as jnp
from jax import lax
from jax.experimental import pallas as pl
from jax.experimental.pallas import tpu as pltpu
from jax.experimental.pallas import tpu_sc as plsc

_B = 16384
_L = 50
_D_ID = 64
_N_STU = 1000000
_NC = 2
_NS = 16
_NW = _NC * _NS
_ROWS_W = _B // _NW
_CHUNK_B = 2
_CHUNK_I = _CHUNK_B * _L
_NCHUNK = _ROWS_W // _CHUNK_B
_LANE = 16
_NG = _D_ID // _LANE

_SC_PARAMS = pltpu.CompilerParams(use_tc_tiling_on_sc=False)


def _sc_course(course_idx2d, w_course):
    mesh = plsc.VectorSubcoreMesh(core_axis_name="c", subcore_axis_name="s")

    @functools.partial(
        pl.kernel,
        out_type=jax.ShapeDtypeStruct((_B, _D_ID), jnp.float32),
        mesh=mesh,
        scratch_types=[
            pltpu.VMEM((_NCHUNK, _CHUNK_I), jnp.int32),
            pltpu.VMEM((_CHUNK_I, _D_ID), jnp.float32),
            pltpu.VMEM((_CHUNK_I, _D_ID), jnp.float32),
            pltpu.VMEM((_ROWS_W, _D_ID), jnp.float32),
            pltpu.SemaphoreType.DMA,
            pltpu.SemaphoreType.DMA,
        ],
        compiler_params=_SC_PARAMS,
    )
    def k(cidx_hbm, wc_hbm, crs_out, cidx_v, buf_a, buf_b, out_v,
          sem_a, sem_b):
        wid = lax.axis_index("s") * _NC + lax.axis_index("c")

        pltpu.sync_copy(cidx_hbm.at[pl.ds(wid * _NCHUNK, _NCHUNK)], cidx_v)

        def issue(c, buf, sem):
            pltpu.async_copy(wc_hbm.at[cidx_v.at[c]], buf, sem)

        def wait(c, buf, sem):
            pltpu.make_async_copy(wc_hbm.at[cidx_v.at[c]], buf, sem).wait()

        def reduce_chunk(c, buf):
            for r in range(_CHUNK_B):
                accs = [buf[r * _L, pl.ds(g * _LANE, _LANE)]
                        for g in range(_NG)]
                for l in range(1, _L):
                    for g in range(_NG):
                        accs[g] = accs[g] + buf[r * _L + l,
                                                pl.ds(g * _LANE, _LANE)]
                row = c * _CHUNK_B + r
                for g in range(_NG):
                    out_v[row, pl.ds(g * _LANE, _LANE)] = accs[g] * (1.0 / _L)

        issue(0, buf_a, sem_a)
        issue(1, buf_b, sem_b)

        @pl.loop(0, _NCHUNK, step=2)
        def _(c):
            wait(c, buf_a, sem_a)
            reduce_chunk(c, buf_a)

            @pl.when(c + 2 < _NCHUNK)
            def _():
                issue(c + 2, buf_a, sem_a)

            wait(c + 1, buf_b, sem_b)
            reduce_chunk(c + 1, buf_b)

            @pl.when(c + 3 < _NCHUNK)
            def _():
                issue(c + 3, buf_b, sem_b)

        pltpu.sync_copy(out_v, crs_out.at[pl.ds(wid * _ROWS_W, _ROWS_W)])

    return k(course_idx2d, w_course)


def _sc_student(student_idx2d, ws_rm):
    mesh = plsc.VectorSubcoreMesh(core_axis_name="c", subcore_axis_name="s")

    @functools.partial(
        pl.kernel,
        out_type=jax.ShapeDtypeStruct((_B, _D_ID), jnp.float32),
        mesh=mesh,
        scratch_types=[
            pltpu.VMEM((4, 128), jnp.int32),
            pltpu.VMEM((128, _D_ID), jnp.float32),
            pltpu.SemaphoreType.DMA,
        ],
        compiler_params=_SC_PARAMS,
    )
    def k(sidx_hbm, ws_hbm, stu_out, sidx_v, srows_v, sem):
        wid = lax.axis_index("s") * _NC + lax.axis_index("c")
        pltpu.sync_copy(sidx_hbm.at[pl.ds(wid * 4, 4)], sidx_v)
        for j in range(4):
            pltpu.async_copy(ws_hbm.at[sidx_v.at[j]], srows_v, sem).wait()
            pltpu.sync_copy(
                srows_v, stu_out.at[pl.ds(wid * _ROWS_W + j * 128, 128)])

    return k(student_idx2d, ws_rm)


_TBLK = 512


def _tc_transpose(wst):
    grid = (pl.cdiv(_N_STU, _TBLK),)

    def body(x_ref, o_ref):
        o_ref[...] = x_ref[...].T

    return pl.pallas_call(
        body,
        grid=grid,
        in_specs=[pl.BlockSpec((_D_ID, _TBLK), lambda j: (0, j))],
        out_specs=pl.BlockSpec((_TBLK, _D_ID), lambda j: (j, 0)),
        out_shape=jax.ShapeDtypeStruct((_N_STU, _D_ID), jnp.float32),
    )(wst)


_BLK = 512


def _tc_body(hist_ref, term_ref, col_ref, maj_ref, cc_ref,
             wh_ref, bh_ref, wt_ref, wcol_ref, wmaj_ref, wcc_ref, bcc_ref,
             out_ref):
    hist = hist_ref[...]
    hproj = (jnp.dot(hist, wh_ref[...], preferred_element_type=jnp.float32)
             * (1.0 / _L) + bh_ref[...])

    term = term_ref[...]
    bins = lax.broadcasted_iota(jnp.int32, (1, 64), 1)
    counts = jnp.zeros((_BLK, 64), jnp.float32)
    for l in range(_L):
        counts = counts + (term[:, l:l + 1] == bins).astype(jnp.float32)
    term_mean = jnp.dot(counts, wt_ref[...],
                        preferred_element_type=jnp.float32) * (1.0 / _L)

    col_oh = (col_ref[...] == lax.broadcasted_iota(jnp.int32, (1, 32), 1)
              ).astype(jnp.float32)
    e_col = jnp.dot(col_oh, wcol_ref[...], preferred_element_type=jnp.float32)

    maj_oh = (maj_ref[...] == lax.broadcasted_iota(jnp.int32, (1, 256), 1)
              ).astype(jnp.float32)
    e_maj = jnp.dot(maj_oh, wmaj_ref[...], preferred_element_type=jnp.float32)

    cc = cc_ref[...]
    wcc = wcc_ref[...]
    c_proj = cc[:, 0:1] * wcc[0:1, :] + cc[:, 1:2] * wcc[1:2, :] + bcc_ref[...]

    out_ref[...] = jnp.concatenate(
        [term_mean, hproj, e_col, e_maj, c_proj], axis=1)


def _tc_dense(hist_flat, term_idx, col2, maj2, course_cont,
              wh_rep, bh2, wt_pad, wcol_pad, wmaj_pad, w_cc, bcc2):
    grid = (_B // _BLK,)
    full = lambda shape: pl.BlockSpec(shape, lambda i: (0, 0))
    blk = lambda minor: pl.BlockSpec((_BLK, minor), lambda i: (i, 0))
    return pl.pallas_call(
        _tc_body,
        grid=grid,
        in_specs=[
            blk(_L * 16),
            blk(_L),
            blk(1),
            blk(1),
            blk(2),
            full((_L * 16, 16)),
            full((1, 16)),
            full((64, 32)),
            full((32, 16)),
            full((256, 16)),
            full((2, 16)),
            full((1, 16)),
        ],
        out_specs=blk(96),
        out_shape=jax.ShapeDtypeStruct((_B, 96), jnp.float32),
    )(hist_flat, term_idx, col2, maj2, course_cont,
      wh_rep, bh2, wt_pad, wcol_pad, wmaj_pad, w_cc, bcc2)


def kernel(student_idx, course_idx, term_idx, college_idx, major_idx,
           hist_cont, course_cont,
           W_student, W_course, W_term, W_college, W_major,
           W_hist, b_hist, W_cc, b_cc):
    cidx2 = course_idx.astype(jnp.int32).reshape(_B * _L // _CHUNK_I, _CHUNK_I)
    crs_mean = _sc_course(cidx2, W_course)

    ws_rm = _tc_transpose(W_student.T)
    stu = _sc_student(student_idx.astype(jnp.int32).reshape(128, 128), ws_rm)

    hist_flat = hist_cont.reshape(_B, _L * 16)
    wh_rep = jnp.tile(W_hist, (_L, 1))
    wt_pad = jnp.zeros((64, 32), jnp.float32).at[:51].set(W_term)
    wcol_pad = jnp.zeros((32, 16), jnp.float32).at[:31].set(W_college)
    wmaj_pad = jnp.zeros((256, 16), jnp.float32).at[:201].set(W_major)
    tc = _tc_dense(hist_flat, term_idx.astype(jnp.int32),
                   college_idx.astype(jnp.int32).reshape(_B, 1),
                   major_idx.astype(jnp.int32).reshape(_B, 1),
                   course_cont, wh_rep, b_hist.reshape(1, 16),
                   wt_pad, wcol_pad, wmaj_pad, W_cc, b_cc.reshape(1, 16))

    return jnp.concatenate([crs_mean, tc[:, :48], stu, tc[:, 48:]], axis=1)


# speedup vs baseline: 1.5736x; 1.5736x over previous
import functools

import jax
import jax.numpy as jnp
from jax import lax
from jax.experimental import pallas as pl
from jax.experimental.pallas import tpu as pltpu
from jax.experimental.pallas import tpu_sc as plsc

_B = 16384
_L = 50
_D_ID = 64
_N_STU = 1000000
_NC = 2
_NS = 16
_NW = _NC * _NS
_ROWS_W = _B // _NW
_CHUNK_B = 2
_CHUNK_I = _CHUNK_B * _L
_NCHUNK = _ROWS_W // _CHUNK_B
_LANE = 16
_NG = _D_ID // _LANE

_SC_PARAMS = pltpu.CompilerParams(use_tc_tiling_on_sc=False)


def _sc_course(course_idx2d, w_course):
    mesh = plsc.VectorSubcoreMesh(core_axis_name="c", subcore_axis_name="s")

    @functools.partial(
        pl.kernel,
        out_type=jax.ShapeDtypeStruct((_B, _D_ID), jnp.float32),
        mesh=mesh,
        scratch_types=[
            pltpu.VMEM((_NCHUNK, _CHUNK_I), jnp.int32),
            pltpu.VMEM((_CHUNK_I, _D_ID), jnp.float32),
            pltpu.VMEM((_CHUNK_I, _D_ID), jnp.float32),
            pltpu.VMEM((_CHUNK_I, _D_ID), jnp.float32),
            pltpu.VMEM((_CHUNK_I, _D_ID), jnp.float32),
            pltpu.VMEM((_ROWS_W, _D_ID), jnp.float32),
            pltpu.SemaphoreType.DMA,
            pltpu.SemaphoreType.DMA,
            pltpu.SemaphoreType.DMA,
            pltpu.SemaphoreType.DMA,
        ],
        compiler_params=_SC_PARAMS,
    )
    def k(cidx_hbm, wc_hbm, crs_out, cidx_v, buf_a, buf_b, buf_c, buf_d,
          out_v, sem_a, sem_b, sem_c, sem_d):
        wid = lax.axis_index("s") * _NC + lax.axis_index("c")

        pltpu.sync_copy(cidx_hbm.at[pl.ds(wid * _NCHUNK, _NCHUNK)], cidx_v)

        def issue(c, buf, sem):
            pltpu.async_copy(wc_hbm.at[cidx_v.at[c]], buf, sem)

        def wait(c, buf, sem):
            pltpu.make_async_copy(wc_hbm.at[cidx_v.at[c]], buf, sem).wait()

        def reduce_chunk(c, buf):
            for r in range(_CHUNK_B):
                accs = [buf[r * _L, pl.ds(g * _LANE, _LANE)]
                        for g in range(_NG)]
                for l in range(1, _L):
                    for g in range(_NG):
                        accs[g] = accs[g] + buf[r * _L + l,
                                                pl.ds(g * _LANE, _LANE)]
                row = c * _CHUNK_B + r
                for g in range(_NG):
                    out_v[row, pl.ds(g * _LANE, _LANE)] = accs[g] * (1.0 / _L)

        bufs = (buf_a, buf_b, buf_c, buf_d)
        sems = (sem_a, sem_b, sem_c, sem_d)
        nbuf = 4

        for b in range(nbuf):
            issue(b, bufs[b], sems[b])

        @pl.loop(0, _NCHUNK, step=nbuf)
        def _(c):
            for b in range(nbuf):
                wait(c + b, bufs[b], sems[b])
                reduce_chunk(c + b, bufs[b])

                @pl.when(c + b + nbuf < _NCHUNK)
                def _(_b=b):
                    issue(c + _b + nbuf, bufs[_b], sems[_b])

        pltpu.sync_copy(out_v, crs_out.at[pl.ds(wid * _ROWS_W, _ROWS_W)])

    return k(course_idx2d, w_course)


def _sc_student(student_idx2d, ws_rm):
    mesh = plsc.VectorSubcoreMesh(core_axis_name="c", subcore_axis_name="s")

    @functools.partial(
        pl.kernel,
        out_type=jax.ShapeDtypeStruct((_B, _D_ID), jnp.float32),
        mesh=mesh,
        scratch_types=[
            pltpu.VMEM((4, 128), jnp.int32),
            pltpu.VMEM((128, _D_ID), jnp.float32),
            pltpu.SemaphoreType.DMA,
        ],
        compiler_params=_SC_PARAMS,
    )
    def k(sidx_hbm, ws_hbm, stu_out, sidx_v, srows_v, sem):
        wid = lax.axis_index("s") * _NC + lax.axis_index("c")
        pltpu.sync_copy(sidx_hbm.at[pl.ds(wid * 4, 4)], sidx_v)
        for j in range(4):
            pltpu.async_copy(ws_hbm.at[sidx_v.at[j]], srows_v, sem).wait()
            pltpu.sync_copy(
                srows_v, stu_out.at[pl.ds(wid * _ROWS_W + j * 128, 128)])

    return k(student_idx2d, ws_rm)


_TBLK = 2048


def _tc_transpose(wst, eye):
    grid = (pl.cdiv(_N_STU, _TBLK),)

    def body(x_ref, e_ref, o_ref):
        o_ref[...] = jax.lax.dot_general(
            x_ref[...], e_ref[...], (((0,), (0,)), ((), ())),
            preferred_element_type=jnp.float32)

    return pl.pallas_call(
        body,
        grid=grid,
        in_specs=[pl.BlockSpec((_D_ID, _TBLK), lambda j: (0, j)),
                  pl.BlockSpec((_D_ID, _D_ID), lambda j: (0, 0))],
        out_specs=pl.BlockSpec((_TBLK, _D_ID), lambda j: (j, 0)),
        out_shape=jax.ShapeDtypeStruct((_N_STU, _D_ID), jnp.float32),
    )(wst, eye)


_BLK = 512


def _tc_body(hist_ref, term_ref, col_ref, maj_ref, cc_ref,
             wh_ref, bh_ref, wt_ref, wcol_ref, wmaj_ref, wcc_ref, bcc_ref,
             out_ref):
    hist = hist_ref[...]
    hproj = (jnp.dot(hist, wh_ref[...], preferred_element_type=jnp.float32)
             * (1.0 / _L) + bh_ref[...])

    term = term_ref[...]
    bins = lax.broadcasted_iota(jnp.int32, (1, 64), 1)
    counts = jnp.zeros((_BLK, 64), jnp.float32)
    for l in range(_L):
        counts = counts + (term[:, l:l + 1] == bins).astype(jnp.float32)
    term_mean = jnp.dot(counts, wt_ref[...],
                        preferred_element_type=jnp.float32) * (1.0 / _L)

    col_oh = (col_ref[...] == lax.broadcasted_iota(jnp.int32, (1, 32), 1)
              ).astype(jnp.float32)
    e_col = jnp.dot(col_oh, wcol_ref[...], preferred_element_type=jnp.float32)

    maj_oh = (maj_ref[...] == lax.broadcasted_iota(jnp.int32, (1, 256), 1)
              ).astype(jnp.float32)
    e_maj = jnp.dot(maj_oh, wmaj_ref[...], preferred_element_type=jnp.float32)

    cc = cc_ref[...]
    wcc = wcc_ref[...]
    c_proj = cc[:, 0:1] * wcc[0:1, :] + cc[:, 1:2] * wcc[1:2, :] + bcc_ref[...]

    out_ref[...] = jnp.concatenate(
        [term_mean, hproj, e_col, e_maj, c_proj], axis=1)


def _tc_dense(hist_flat, term_idx, col2, maj2, course_cont,
              wh_rep, bh2, wt_pad, wcol_pad, wmaj_pad, w_cc, bcc2):
    grid = (_B // _BLK,)
    full = lambda shape: pl.BlockSpec(shape, lambda i: (0, 0))
    blk = lambda minor: pl.BlockSpec((_BLK, minor), lambda i: (i, 0))
    return pl.pallas_call(
        _tc_body,
        grid=grid,
        in_specs=[
            blk(_L * 16),
            blk(_L),
            blk(1),
            blk(1),
            blk(2),
            full((_L * 16, 16)),
            full((1, 16)),
            full((64, 32)),
            full((32, 16)),
            full((256, 16)),
            full((2, 16)),
            full((1, 16)),
        ],
        out_specs=blk(96),
        out_shape=jax.ShapeDtypeStruct((_B, 96), jnp.float32),
    )(hist_flat, term_idx, col2, maj2, course_cont,
      wh_rep, bh2, wt_pad, wcol_pad, wmaj_pad, w_cc, bcc2)


def kernel(student_idx, course_idx, term_idx, college_idx, major_idx,
           hist_cont, course_cont,
           W_student, W_course, W_term, W_college, W_major,
           W_hist, b_hist, W_cc, b_cc):
    cidx2 = course_idx.astype(jnp.int32).reshape(_B * _L // _CHUNK_I, _CHUNK_I)
    crs_mean = _sc_course(cidx2, W_course)

    ws_rm = _tc_transpose(W_student.T, jnp.eye(_D_ID, dtype=jnp.float32))
    stu = _sc_student(student_idx.astype(jnp.int32).reshape(128, 128), ws_rm)

    hist_flat = hist_cont.reshape(_B, _L * 16)
    wh_rep = jnp.tile(W_hist, (_L, 1))
    wt_pad = jnp.zeros((64, 32), jnp.float32).at[:51].set(W_term)
    wcol_pad = jnp.zeros((32, 16), jnp.float32).at[:31].set(W_college)
    wmaj_pad = jnp.zeros((256, 16), jnp.float32).at[:201].set(W_major)
    tc = _tc_dense(hist_flat, term_idx.astype(jnp.int32),
                   college_idx.astype(jnp.int32).reshape(_B, 1),
                   major_idx.astype(jnp.int32).reshape(_B, 1),
                   course_cont, wh_rep, b_hist.reshape(1, 16),
                   wt_pad, wcol_pad, wmaj_pad, W_cc, b_cc.reshape(1, 16))

    return jnp.concatenate([crs_mean, tc[:, :48], stu, tc[:, 48:]], axis=1)


# speedup vs baseline: 2.0608x; 1.3095x over previous
import functools

import jax
import jax.numpy as jnp
from jax import lax
from jax.experimental import pallas as pl
from jax.experimental.pallas import tpu as pltpu
from jax.experimental.pallas import tpu_sc as plsc

_B = 16384
_L = 50
_D_ID = 64
_N_STU = 1000000
_NC = 2
_NS = 16
_NW = _NC * _NS
_ROWS_W = _B // _NW
_CHUNK_B = 2
_CHUNK_I = _CHUNK_B * _L
_NCHUNK = _ROWS_W // _CHUNK_B
_LANE = 16
_NG = _D_ID // _LANE

_SC_PARAMS = pltpu.CompilerParams(use_tc_tiling_on_sc=False)


def _sc_course(course_idx2d, w_course):
    mesh = plsc.VectorSubcoreMesh(core_axis_name="c", subcore_axis_name="s")

    @functools.partial(
        pl.kernel,
        out_type=jax.ShapeDtypeStruct((_B, _D_ID), jnp.float32),
        mesh=mesh,
        scratch_types=[
            pltpu.VMEM((_NCHUNK, _CHUNK_I), jnp.int32),
            pltpu.VMEM((_CHUNK_I, _D_ID), jnp.float32),
            pltpu.VMEM((_CHUNK_I, _D_ID), jnp.float32),
            pltpu.VMEM((_CHUNK_I, _D_ID), jnp.float32),
            pltpu.VMEM((_CHUNK_I, _D_ID), jnp.float32),
            pltpu.VMEM((_ROWS_W, _D_ID), jnp.float32),
            pltpu.SemaphoreType.DMA,
            pltpu.SemaphoreType.DMA,
            pltpu.SemaphoreType.DMA,
            pltpu.SemaphoreType.DMA,
        ],
        compiler_params=_SC_PARAMS,
    )
    def k(cidx_hbm, wc_hbm, crs_out, cidx_v, buf_a, buf_b, buf_c, buf_d,
          out_v, sem_a, sem_b, sem_c, sem_d):
        wid = lax.axis_index("s") * _NC + lax.axis_index("c")

        pltpu.sync_copy(cidx_hbm.at[pl.ds(wid * _NCHUNK, _NCHUNK)], cidx_v)

        def issue(c, buf, sem):
            pltpu.async_copy(wc_hbm.at[cidx_v.at[c]], buf, sem)

        def wait(c, buf, sem):
            pltpu.make_async_copy(wc_hbm.at[cidx_v.at[c]], buf, sem).wait()

        def reduce_chunk(c, buf):
            for r in range(_CHUNK_B):
                accs = [buf[r * _L, pl.ds(g * _LANE, _LANE)]
                        for g in range(_NG)]
                for l in range(1, _L):
                    for g in range(_NG):
                        accs[g] = accs[g] + buf[r * _L + l,
                                                pl.ds(g * _LANE, _LANE)]
                row = c * _CHUNK_B + r
                for g in range(_NG):
                    out_v[row, pl.ds(g * _LANE, _LANE)] = accs[g] * (1.0 / _L)

        bufs = (buf_a, buf_b, buf_c, buf_d)
        sems = (sem_a, sem_b, sem_c, sem_d)
        nbuf = 4

        for b in range(nbuf):
            issue(b, bufs[b], sems[b])

        @pl.loop(0, _NCHUNK, step=nbuf)
        def _(c):
            for b in range(nbuf):
                wait(c + b, bufs[b], sems[b])
                reduce_chunk(c + b, bufs[b])

                @pl.when(c + b + nbuf < _NCHUNK)
                def _(_b=b):
                    issue(c + _b + nbuf, bufs[_b], sems[_b])

        pltpu.sync_copy(out_v, crs_out.at[pl.ds(wid * _ROWS_W, _ROWS_W)])

    return k(course_idx2d, w_course)


def _sc_student(student_idx2d, ws_rm):
    mesh = plsc.VectorSubcoreMesh(core_axis_name="c", subcore_axis_name="s")

    @functools.partial(
        pl.kernel,
        out_type=jax.ShapeDtypeStruct((_B, _D_ID), jnp.float32),
        mesh=mesh,
        scratch_types=[
            pltpu.VMEM((4, 128), jnp.int32),
            pltpu.VMEM((128, _D_ID), jnp.float32),
            pltpu.SemaphoreType.DMA,
        ],
        compiler_params=_SC_PARAMS,
    )
    def k(sidx_hbm, ws_hbm, stu_out, sidx_v, srows_v, sem):
        wid = lax.axis_index("s") * _NC + lax.axis_index("c")
        pltpu.sync_copy(sidx_hbm.at[pl.ds(wid * 4, 4)], sidx_v)
        for j in range(4):
            pltpu.async_copy(ws_hbm.at[sidx_v.at[j]], srows_v, sem).wait()
            pltpu.sync_copy(
                srows_v, stu_out.at[pl.ds(wid * _ROWS_W + j * 128, 128)])

    return k(student_idx2d, ws_rm)


_BLK = 512


def _tc_body(hist_ref, term_ref, col_ref, maj_ref, cc_ref,
             wh_ref, bh_ref, wt_ref, wcol_ref, wmaj_ref, wcc_ref, bcc_ref,
             out_ref):
    hist = hist_ref[...]
    hproj = (jnp.dot(hist, wh_ref[...], preferred_element_type=jnp.float32)
             * (1.0 / _L) + bh_ref[...])

    term = term_ref[...]
    bins = lax.broadcasted_iota(jnp.int32, (1, 64), 1)
    counts = jnp.zeros((_BLK, 64), jnp.float32)
    for l in range(_L):
        counts = counts + (term[:, l:l + 1] == bins).astype(jnp.float32)
    term_mean = jnp.dot(counts, wt_ref[...],
                        preferred_element_type=jnp.float32) * (1.0 / _L)

    col_oh = (col_ref[...] == lax.broadcasted_iota(jnp.int32, (1, 32), 1)
              ).astype(jnp.float32)
    e_col = jnp.dot(col_oh, wcol_ref[...], preferred_element_type=jnp.float32)

    maj_oh = (maj_ref[...] == lax.broadcasted_iota(jnp.int32, (1, 256), 1)
              ).astype(jnp.float32)
    e_maj = jnp.dot(maj_oh, wmaj_ref[...], preferred_element_type=jnp.float32)

    cc = cc_ref[...]
    wcc = wcc_ref[...]
    c_proj = cc[:, 0:1] * wcc[0:1, :] + cc[:, 1:2] * wcc[1:2, :] + bcc_ref[...]

    out_ref[...] = jnp.concatenate(
        [term_mean, hproj, e_col, e_maj, c_proj], axis=1)


def _tc_dense(hist_flat, term_idx, col2, maj2, course_cont,
              wh_rep, bh2, wt_pad, wcol_pad, wmaj_pad, w_cc, bcc2):
    grid = (_B // _BLK,)
    full = lambda shape: pl.BlockSpec(shape, lambda i: (0, 0))
    blk = lambda minor: pl.BlockSpec((_BLK, minor), lambda i: (i, 0))
    return pl.pallas_call(
        _tc_body,
        grid=grid,
        in_specs=[
            blk(_L * 16),
            blk(_L),
            blk(1),
            blk(1),
            blk(2),
            full((_L * 16, 16)),
            full((1, 16)),
            full((64, 32)),
            full((32, 16)),
            full((256, 16)),
            full((2, 16)),
            full((1, 16)),
        ],
        out_specs=blk(96),
        out_shape=jax.ShapeDtypeStruct((_B, 96), jnp.float32),
    )(hist_flat, term_idx, col2, maj2, course_cont,
      wh_rep, bh2, wt_pad, wcol_pad, wmaj_pad, w_cc, bcc2)


def kernel(student_idx, course_idx, term_idx, college_idx, major_idx,
           hist_cont, course_cont,
           W_student, W_course, W_term, W_college, W_major,
           W_hist, b_hist, W_cc, b_cc):
    cidx2 = course_idx.astype(jnp.int32).reshape(_B * _L // _CHUNK_I, _CHUNK_I)
    crs_mean = _sc_course(cidx2, W_course)

    stu = _sc_student(student_idx.astype(jnp.int32).reshape(128, 128),
                      W_student)

    hist_flat = hist_cont.reshape(_B, _L * 16)
    wh_rep = jnp.tile(W_hist, (_L, 1))
    wt_pad = jnp.zeros((64, 32), jnp.float32).at[:51].set(W_term)
    wcol_pad = jnp.zeros((32, 16), jnp.float32).at[:31].set(W_college)
    wmaj_pad = jnp.zeros((256, 16), jnp.float32).at[:201].set(W_major)
    tc = _tc_dense(hist_flat, term_idx.astype(jnp.int32),
                   college_idx.astype(jnp.int32).reshape(_B, 1),
                   major_idx.astype(jnp.int32).reshape(_B, 1),
                   course_cont, wh_rep, b_hist.reshape(1, 16),
                   wt_pad, wcol_pad, wmaj_pad, W_cc, b_cc.reshape(1, 16))

    return jnp.concatenate([crs_mean, tc[:, :48], stu, tc[:, 48:]], axis=1)


# speedup vs baseline: 2.0804x; 1.0096x over previous
import functools

import jax
import jax.numpy as jnp
from jax import lax
from jax.experimental import pallas as pl
from jax.experimental.pallas import tpu as pltpu
from jax.experimental.pallas import tpu_sc as plsc

_B = 16384
_L = 50
_D_ID = 64
_N_STU = 1000000
_NC = 2
_NS = 16
_NW = _NC * _NS
_ROWS_W = _B // _NW
_CHUNK_B = 2
_CHUNK_I = _CHUNK_B * _L
_NCHUNK = _ROWS_W // _CHUNK_B
_LANE = 16
_NG = _D_ID // _LANE

_SC_PARAMS = pltpu.CompilerParams(use_tc_tiling_on_sc=False)


def _sc_course(course_idx2d, w_course):
    mesh = plsc.VectorSubcoreMesh(core_axis_name="c", subcore_axis_name="s")

    @functools.partial(
        pl.kernel,
        out_type=jax.ShapeDtypeStruct((_B // 2, 2 * _D_ID), jnp.float32),
        mesh=mesh,
        scratch_types=[
            pltpu.VMEM((_NCHUNK, _CHUNK_I), jnp.int32),
            pltpu.VMEM((_CHUNK_I, _D_ID), jnp.float32),
            pltpu.VMEM((_CHUNK_I, _D_ID), jnp.float32),
            pltpu.VMEM((_CHUNK_I, _D_ID), jnp.float32),
            pltpu.VMEM((_CHUNK_I, _D_ID), jnp.float32),
            pltpu.VMEM((_NCHUNK, 2 * _D_ID), jnp.float32),
            pltpu.SemaphoreType.DMA,
            pltpu.SemaphoreType.DMA,
            pltpu.SemaphoreType.DMA,
            pltpu.SemaphoreType.DMA,
        ],
        compiler_params=_SC_PARAMS,
    )
    def k(cidx_hbm, wc_hbm, crs_out, cidx_v, buf_a, buf_b, buf_c, buf_d,
          out_v, sem_a, sem_b, sem_c, sem_d):
        wid = lax.axis_index("s") * _NC + lax.axis_index("c")

        pltpu.sync_copy(cidx_hbm.at[pl.ds(wid * _NCHUNK, _NCHUNK)], cidx_v)

        def issue(c, buf, sem):
            pltpu.async_copy(wc_hbm.at[cidx_v.at[c]], buf, sem)

        def wait(c, buf, sem):
            pltpu.make_async_copy(wc_hbm.at[cidx_v.at[c]], buf, sem).wait()

        def reduce_chunk(c, buf):
            for r in range(_CHUNK_B):
                accs = [buf[r * _L, pl.ds(g * _LANE, _LANE)]
                        for g in range(_NG)]
                for l in range(1, _L):
                    for g in range(_NG):
                        accs[g] = accs[g] + buf[r * _L + l,
                                                pl.ds(g * _LANE, _LANE)]
                for g in range(_NG):
                    out_v[c, pl.ds(r * _D_ID + g * _LANE, _LANE)] = (
                        accs[g] * (1.0 / _L))

        bufs = (buf_a, buf_b, buf_c, buf_d)
        sems = (sem_a, sem_b, sem_c, sem_d)
        nbuf = 4

        for b in range(nbuf):
            issue(b, bufs[b], sems[b])

        @pl.loop(0, _NCHUNK, step=nbuf)
        def _(c):
            for b in range(nbuf):
                wait(c + b, bufs[b], sems[b])
                reduce_chunk(c + b, bufs[b])

                @pl.when(c + b + nbuf < _NCHUNK)
                def _(_b=b):
                    issue(c + _b + nbuf, bufs[_b], sems[_b])

        pltpu.sync_copy(out_v, crs_out.at[pl.ds(wid * _NCHUNK, _NCHUNK)])

    return k(course_idx2d, w_course)


def _sc_student(jidx2d, ws128):
    mesh = plsc.VectorSubcoreMesh(core_axis_name="c", subcore_axis_name="s")

    @functools.partial(
        pl.kernel,
        out_type=jax.ShapeDtypeStruct((_B, 2 * _D_ID), jnp.float32),
        mesh=mesh,
        scratch_types=[
            pltpu.VMEM((4, 128), jnp.int32),
            pltpu.VMEM((128, 2 * _D_ID), jnp.float32),
            pltpu.SemaphoreType.DMA,
        ],
        compiler_params=_SC_PARAMS,
    )
    def k(jidx_hbm, ws_hbm, stu_out, jidx_v, srows_v, sem):
        wid = lax.axis_index("s") * _NC + lax.axis_index("c")
        pltpu.sync_copy(jidx_hbm.at[pl.ds(wid * 4, 4)], jidx_v)
        for j in range(4):
            pltpu.async_copy(ws_hbm.at[jidx_v.at[j]], srows_v, sem).wait()
            pltpu.sync_copy(
                srows_v, stu_out.at[pl.ds(wid * _ROWS_W + j * 128, 128)])

    return k(jidx2d, ws128)


_BLK = 512


def _tc_body(hist_ref, term_ref, col_ref, maj_ref, cc_ref, stu_ref, par_ref,
             wh_ref, bh_ref, wt_ref, wcol_ref, wmaj_ref, wcc_ref, bcc_ref,
             out_ref):
    hist = hist_ref[...]
    hproj = (jnp.dot(hist, wh_ref[...], preferred_element_type=jnp.float32)
             * (1.0 / _L) + bh_ref[...])

    term = term_ref[...]
    bins = lax.broadcasted_iota(jnp.int32, (1, 64), 1)
    counts = jnp.zeros((_BLK, 64), jnp.float32)
    for l in range(_L):
        counts = counts + (term[:, l:l + 1] == bins).astype(jnp.float32)
    term_mean = jnp.dot(counts, wt_ref[...],
                        preferred_element_type=jnp.float32) * (1.0 / _L)

    stu128 = stu_ref[...]
    par = par_ref[...]
    e_stu = jnp.where(par == 1, stu128[:, _D_ID:], stu128[:, :_D_ID])

    col_oh = (col_ref[...] == lax.broadcasted_iota(jnp.int32, (1, 32), 1)
              ).astype(jnp.float32)
    e_col = jnp.dot(col_oh, wcol_ref[...], preferred_element_type=jnp.float32)

    maj_oh = (maj_ref[...] == lax.broadcasted_iota(jnp.int32, (1, 256), 1)
              ).astype(jnp.float32)
    e_maj = jnp.dot(maj_oh, wmaj_ref[...], preferred_element_type=jnp.float32)

    cc = cc_ref[...]
    wcc = wcc_ref[...]
    c_proj = cc[:, 0:1] * wcc[0:1, :] + cc[:, 1:2] * wcc[1:2, :] + bcc_ref[...]

    out_ref[...] = jnp.concatenate(
        [term_mean, hproj, e_stu, e_col, e_maj, c_proj], axis=1)


def _tc_dense(hist_flat, term_idx, col2, maj2, course_cont, stu128, par2,
              wh_rep, bh2, wt_pad, wcol_pad, wmaj_pad, w_cc, bcc2):
    grid = (_B // _BLK,)
    full = lambda shape: pl.BlockSpec(shape, lambda i: (0, 0))
    blk = lambda minor: pl.BlockSpec((_BLK, minor), lambda i: (i, 0))
    return pl.pallas_call(
        _tc_body,
        grid=grid,
        in_specs=[
            blk(_L * 16),
            blk(_L),
            blk(1),
            blk(1),
            blk(2),
            blk(128),
            blk(1),
            full((_L * 16, 16)),
            full((1, 16)),
            full((64, 32)),
            full((32, 16)),
            full((256, 16)),
            full((2, 16)),
            full((1, 16)),
        ],
        out_specs=blk(160),
        out_shape=jax.ShapeDtypeStruct((_B, 160), jnp.float32),
    )(hist_flat, term_idx, col2, maj2, course_cont, stu128, par2,
      wh_rep, bh2, wt_pad, wcol_pad, wmaj_pad, w_cc, bcc2)


def kernel(student_idx, course_idx, term_idx, college_idx, major_idx,
           hist_cont, course_cont,
           W_student, W_course, W_term, W_college, W_major,
           W_hist, b_hist, W_cc, b_cc):
    cidx2 = course_idx.astype(jnp.int32).reshape(_B * _L // _CHUNK_I, _CHUNK_I)
    crs128 = _sc_course(cidx2, W_course)
    crs_mean = crs128.reshape(_B, _D_ID)

    sidx = student_idx.astype(jnp.int32)
    ws128 = W_student[:_N_STU].reshape(_N_STU // 2, 2 * _D_ID)
    stu128 = _sc_student((sidx // 2).reshape(128, 128), ws128)

    hist_flat = hist_cont.reshape(_B, _L * 16)
    wh_rep = jnp.tile(W_hist, (_L, 1))
    wt_pad = jnp.zeros((64, 32), jnp.float32).at[:51].set(W_term)
    wcol_pad = jnp.zeros((32, 16), jnp.float32).at[:31].set(W_college)
    wmaj_pad = jnp.zeros((256, 16), jnp.float32).at[:201].set(W_major)
    tc = _tc_dense(hist_flat, term_idx.astype(jnp.int32),
                   college_idx.astype(jnp.int32).reshape(_B, 1),
                   major_idx.astype(jnp.int32).reshape(_B, 1),
                   course_cont, stu128, (sidx % 2).reshape(_B, 1),
                   wh_rep, b_hist.reshape(1, 16),
                   wt_pad, wcol_pad, wmaj_pad, W_cc, b_cc.reshape(1, 16))

    return jnp.concatenate([crs_mean, tc], axis=1)
